# baseline (speedup 1.0000x reference)
import functools

import jax
import jax.numpy as jnp
from jax import lax
from jax.experimental import pallas as pl
from jax.experimental.pallas import tpu as pltpu
from jax.experimental.pallas import tpu_sc as plsc

N_NODES = 10000
D = 128
OUTD = 64
E = 320000

NPAD = 10240
EPAD = 327680
EPW = EPAD // 32
EB = 4096
NBLK = EPAD // EB
FPT = 4
BLK = 1024
V = 16


def _sc_compiler_params():
    import dataclasses
    cp = pltpu.CompilerParams()
    if "needs_layout_passes" in pltpu.CompilerParams.__dataclass_fields__:
        cp = dataclasses.replace(cp, needs_layout_passes=False)
    return cp


@functools.cache
def _get_deg_kernel():
    mesh = plsc.VectorSubcoreMesh(core_axis_name="c", subcore_axis_name="s")

    @functools.partial(
        pl.kernel,
        out_type=jax.ShapeDtypeStruct((32, NPAD), jnp.float32),
        mesh=mesh,
        scratch_types=[
            pltpu.VMEM((EPW,), jnp.int32),
            pltpu.VMEM((NPAD,), jnp.float32),
        ],
        compiler_params=_sc_compiler_params(),
    )
    def _deg_kernel(dst_hbm, out_hbm, dstb, degrow):
        cid = lax.axis_index("c")
        sid = lax.axis_index("s")
        w = cid * 16 + sid
        pltpu.sync_copy(dst_hbm.at[pl.ds(w * EPW, EPW)], dstb)

        @pl.loop(0, NPAD, step=V)
        def _(i):
            degrow[pl.ds(i, V)] = jnp.zeros((V,), jnp.float32)

        ones = jnp.full((V,), 1.0, jnp.float32)

        @pl.loop(0, EPW, step=V)
        def _(j):
            d16 = dstb[pl.ds(j, V)]
            plsc.addupdate_scatter(degrow, [d16], ones)

        pltpu.sync_copy(degrow, out_hbm.at[w])

    return _deg_kernel


@functools.cache
def _get_scatter_kernel():
    mesh = plsc.VectorSubcoreMesh(core_axis_name="c", subcore_axis_name="s")

    @functools.partial(
        pl.kernel,
        out_type=jax.ShapeDtypeStruct((D, NPAD), jnp.float32),
        mesh=mesh,
        scratch_types=(
            [pltpu.VMEM((NPAD,), jnp.float32)] * FPT
            + [pltpu.VMEM((NPAD,), jnp.float32)] * FPT
            + [pltpu.VMEM((EB,), jnp.int32)] * 4
            + [pltpu.SemaphoreType.DMA] * 4
        ),
        compiler_params=_sc_compiler_params(),
    )
    def _scatter_kernel(gt_hbm, src_hbm, dst_hbm, out_hbm, *rest):
        g = rest[:FPT]
        a = rest[FPT:2 * FPT]
        sb = rest[2 * FPT:2 * FPT + 2]
        db = rest[2 * FPT + 2:2 * FPT + 4]
        sem_s = rest[2 * FPT + 4:2 * FPT + 6]
        sem_d = rest[2 * FPT + 6:]
        cid = lax.axis_index("c")
        sid = lax.axis_index("s")
        fbase = FPT * (cid * 16 + sid)

        def blk_start(b, p):
            pltpu.async_copy(src_hbm.at[pl.ds(b * EB, EB)], sb[p], sem_s[p])
            pltpu.async_copy(dst_hbm.at[pl.ds(b * EB, EB)], db[p], sem_d[p])

        def blk_wait(b, p):
            pltpu.make_async_copy(src_hbm.at[pl.ds(b * EB, EB)], sb[p],
                                  sem_s[p]).wait()
            pltpu.make_async_copy(dst_hbm.at[pl.ds(b * EB, EB)], db[p],
                                  sem_d[p]).wait()

        for f in range(FPT):
            pltpu.sync_copy(gt_hbm.at[fbase + f], g[f])

        for f in range(FPT):
            @pl.loop(0, NPAD, step=V)
            def _(i, f=f):
                a[f][pl.ds(i, V)] = jnp.zeros((V,), jnp.float32)

        blk_start(0, 0)
        blk_start(1, 1)

        @pl.loop(0, NBLK, step=2)
        def _(nb):
            for p in range(2):
                b = nb + p
                blk_wait(b, p)

                @pl.loop(0, EB, step=V)
                def _(j, p=p):
                    s16 = sb[p][pl.ds(j, V)]
                    d16 = db[p][pl.ds(j, V)]
                    for f in range(FPT):
                        v = plsc.load_gather(g[f], [s16])
                        plsc.addupdate_scatter(a[f], [d16], v)

                @pl.when(b + 2 < NBLK)
                def _():
                    blk_start(b + 2, p)

        for f in range(FPT):
            pltpu.sync_copy(a[f], out_hbm.at[fbase + f])

    return _scatter_kernel


def _mmt_body(w_ref, x_ref, o_ref):
    o_ref[...] = lax.dot_general(
        w_ref[...], x_ref[...],
        dimension_numbers=(((0,), (1,)), ((), ())),
        preferred_element_type=jnp.float32)


def _tc_matmul_t(W1, x):
    return pl.pallas_call(
        _mmt_body,
        grid=(NPAD // BLK,),
        in_specs=[pl.BlockSpec((D, D), lambda i: (0, 0)),
                  pl.BlockSpec((BLK, D), lambda i: (i, 0))],
        out_specs=pl.BlockSpec((D, BLK), lambda i: (0, i)),
        out_shape=jax.ShapeDtypeStruct((D, NPAD), jnp.float32),
    )(W1, x)


def _prep_body(degp_ref, h1_ref, dinv_ref, g1_ref):
    deg = jnp.sum(degp_ref[...], axis=0, keepdims=True) + 1.0
    dinv = lax.rsqrt(deg)
    dinv_ref[...] = dinv
    g1_ref[...] = h1_ref[...] * dinv


def _tc_prep(degp, h1t):
    return pl.pallas_call(
        _prep_body,
        grid=(NPAD // BLK,),
        in_specs=[pl.BlockSpec((32, BLK), lambda i: (0, i)),
                  pl.BlockSpec((D, BLK), lambda i: (0, i))],
        out_specs=[pl.BlockSpec((1, BLK), lambda i: (0, i)),
                   pl.BlockSpec((D, BLK), lambda i: (0, i))],
        out_shape=[jax.ShapeDtypeStruct((1, NPAD), jnp.float32),
                   jax.ShapeDtypeStruct((D, NPAD), jnp.float32)],
    )(degp, h1t)


def _mid_body(s_ref, g1_ref, dinv_ref, b1_ref, wlint_ref, blin_ref, w2t_ref,
              g2_ref):
    dinv = dinv_ref[...]
    a = (s_ref[...] + g1_ref[...]) * dinv + b1_ref[...]
    t = jnp.tanh(a)
    l = jnp.dot(wlint_ref[...], t, preferred_element_type=jnp.float32)
    l = l + blin_ref[...]
    h2 = jnp.dot(w2t_ref[...], l, preferred_element_type=jnp.float32)
    g2_ref[...] = h2 * dinv


def _tc_mid(s1, g1, dinv, b1c, WlinT, blinc, W2T):
    return pl.pallas_call(
        _mid_body,
        grid=(NPAD // BLK,),
        in_specs=[pl.BlockSpec((D, BLK), lambda i: (0, i)),
                  pl.BlockSpec((D, BLK), lambda i: (0, i)),
                  pl.BlockSpec((1, BLK), lambda i: (0, i)),
                  pl.BlockSpec((D, 1), lambda i: (0, 0)),
                  pl.BlockSpec((D, D), lambda i: (0, 0)),
                  pl.BlockSpec((D, 1), lambda i: (0, 0)),
                  pl.BlockSpec((D, D), lambda i: (0, 0))],
        out_specs=pl.BlockSpec((D, BLK), lambda i: (0, i)),
        out_shape=jax.ShapeDtypeStruct((D, NPAD), jnp.float32),
    )(s1, g1, dinv, b1c, WlinT, blinc, W2T)


def _fin_body(s_ref, g2_ref, dinv_ref, b2_ref, woutt_ref, bout_ref, o_ref):
    i = pl.program_id(0)
    a = (s_ref[...] + g2_ref[...]) * dinv_ref[...] + b2_ref[...]
    t = jnp.tanh(a)
    o = jnp.dot(woutt_ref[...], t, preferred_element_type=jnp.float32)
    o = o + bout_ref[...]
    cols = lax.broadcasted_iota(jnp.int32, (OUTD, BLK), 1) + i * BLK
    o = jnp.where(cols < N_NODES, o, -jnp.inf)
    m = jnp.max(o, axis=1, keepdims=True)

    @pl.when(i == 0)
    def _():
        o_ref[...] = m

    @pl.when(i != 0)
    def _():
        o_ref[...] = jnp.maximum(o_ref[...], m)


def _tc_final(s2, g2, dinv, b2c, WoutT, boutc):
    return pl.pallas_call(
        _fin_body,
        grid=(NPAD // BLK,),
        in_specs=[pl.BlockSpec((D, BLK), lambda i: (0, i)),
                  pl.BlockSpec((D, BLK), lambda i: (0, i)),
                  pl.BlockSpec((1, BLK), lambda i: (0, i)),
                  pl.BlockSpec((D, 1), lambda i: (0, 0)),
                  pl.BlockSpec((OUTD, D), lambda i: (0, 0)),
                  pl.BlockSpec((OUTD, 1), lambda i: (0, 0))],
        out_specs=pl.BlockSpec((OUTD, 1), lambda i: (0, 0)),
        out_shape=jax.ShapeDtypeStruct((OUTD, 1), jnp.float32),
    )(s2, g2, dinv, b2c, WoutT, boutc)


def kernel(x, edge_index, W1, b1, Wlin, blin, W2, b2, Wout, bout):
    src = jnp.pad(edge_index[0].astype(jnp.int32), (0, EPAD - E))
    dst = jnp.pad(edge_index[1].astype(jnp.int32), (0, EPAD - E),
                  constant_values=N_NODES)
    x_p = jnp.pad(x, ((0, NPAD - N_NODES), (0, 0)))
    b1c = b1.reshape(D, 1)
    blinc = blin.reshape(D, 1)
    b2c = b2.reshape(D, 1)
    boutc = bout.reshape(OUTD, 1)
    WlinT = Wlin.T
    W2T = W2.T
    WoutT = Wout.T

    deg_kernel = _get_deg_kernel()
    scatter_kernel = _get_scatter_kernel()
    degp = deg_kernel(dst)
    h1t = _tc_matmul_t(W1, x_p)
    dinv, g1 = _tc_prep(degp, h1t)
    s1 = scatter_kernel(g1, src, dst)
    g2 = _tc_mid(s1, g1, dinv, b1c, WlinT, blinc, W2T)
    s2 = scatter_kernel(g2, src, dst)
    out = _tc_final(s2, g2, dinv, b2c, WoutT, boutc)
    return out.T


# speedup vs baseline: 1.1774x; 1.1774x over previous
import functools

import jax
import jax.numpy as jnp
from jax import lax
from jax.experimental import pallas as pl
from jax.experimental.pallas import tpu as pltpu
from jax.experimental.pallas import tpu_sc as plsc

N_NODES = 10000
D = 128
OUTD = 64
E = 320000

NPAD = 10240
CHUNK = 128
NW = 32
CPW = 80
NCHUNKS = NW * CPW
EPAD = NCHUNKS * CHUNK
RPW = NPAD // 16
NBUF = 2
NIDX = 4
CPW0 = 132
CPW1 = NCHUNKS // 16 - CPW0
BLK = 1024


@functools.cache
def _get_deg_kernel():
    mesh = plsc.VectorSubcoreMesh(core_axis_name="c", subcore_axis_name="s")

    @functools.partial(
        pl.kernel,
        out_type=jax.ShapeDtypeStruct((2, NPAD, D), jnp.float32),
        mesh=mesh,
        scratch_types=(
            [pltpu.VMEM((2, CHUNK), jnp.int32)] * NIDX
            + [pltpu.VMEM((CHUNK, D), jnp.float32)]
            + [pltpu.VMEM_SHARED((NPAD, D), jnp.float32)]
            + [pltpu.SemaphoreType.DMA] * NIDX
        ),
    )
    def _deg_kernel(eidx_hbm, ones_hbm, zeros_hbm, out_hbm, *rest):
        idx = rest[:NIDX]
        ones_v = rest[NIDX]
        acc_sh = rest[NIDX + 1]
        sem_i = rest[NIDX + 2:]
        cid = lax.axis_index("c")
        sid = lax.axis_index("s")
        w = cid * 16 + sid
        base = w * CPW

        def idx_start(c, q):
            pltpu.async_copy(eidx_hbm.at[pl.ds(2 * (base + c), 2)], idx[q],
                             sem_i[q])

        def idx_wait(c, q):
            pltpu.make_async_copy(eidx_hbm.at[pl.ds(2 * (base + c), 2)],
                                  idx[q], sem_i[q]).wait()

        pltpu.sync_copy(zeros_hbm, acc_sh.at[pl.ds(sid * RPW, RPW)])
        pltpu.sync_copy(ones_hbm, ones_v)
        for q in range(NIDX):
            idx_start(q, q)
        plsc.subcore_barrier()

        @pl.loop(0, CPW, step=NIDX)
        def _(i):
            for q in range(NIDX):
                c = i + q
                idx_wait(c, q)
                pltpu.sync_copy(ones_v, acc_sh.at[idx[q].at[1]], add=True)

                @pl.when(c + NIDX < CPW)
                def _():
                    idx_start(c + NIDX, q)

        plsc.subcore_barrier()
        pltpu.sync_copy(acc_sh.at[pl.ds(sid * RPW, RPW)],
                        out_hbm.at[cid, pl.ds(sid * RPW, RPW)])

    return _deg_kernel


@functools.cache
def _get_scatter_kernel():
    mesh = plsc.VectorSubcoreMesh(core_axis_name="c", subcore_axis_name="s")

    @functools.partial(
        pl.kernel,
        out_type=jax.ShapeDtypeStruct((2, NPAD, D), jnp.float32),
        mesh=mesh,
        scratch_types=(
            [pltpu.VMEM((2, CHUNK), jnp.int32)] * NIDX
            + [pltpu.VMEM((CHUNK, D), jnp.float32)] * NBUF
            + [pltpu.VMEM_SHARED((NPAD, D), jnp.float32)]
            + [pltpu.SemaphoreType.DMA] * (NIDX + NBUF)
        ),
    )
    def _scatter_kernel(g_hbm, eidx_hbm, zeros_hbm, out_hbm, *rest):
        idx = rest[:NIDX]
        rows = rest[NIDX:NIDX + NBUF]
        acc_sh = rest[NIDX + NBUF]
        sem_i = rest[NIDX + NBUF + 1:NIDX + NBUF + 1 + NIDX]
        sem_g = rest[NIDX + NBUF + 1 + NIDX:]
        cid = lax.axis_index("c")
        sid = lax.axis_index("s")
        my_cpw = jnp.where(cid == 0, CPW0, CPW1)
        base = jnp.where(cid == 0, sid * CPW0, 16 * CPW0 + sid * CPW1)

        def idx_start(c, q):
            pltpu.async_copy(eidx_hbm.at[pl.ds(2 * (base + c), 2)], idx[q],
                             sem_i[q])

        def idx_wait(c, q):
            pltpu.make_async_copy(eidx_hbm.at[pl.ds(2 * (base + c), 2)],
                                  idx[q], sem_i[q]).wait()

        def g_start(q, r):
            pltpu.async_copy(g_hbm.at[idx[q].at[0]], rows[r], sem_g[r])

        def g_wait(q, r):
            pltpu.make_async_copy(g_hbm.at[idx[q].at[0]], rows[r],
                                  sem_g[r]).wait()

        pltpu.sync_copy(zeros_hbm, acc_sh.at[pl.ds(sid * RPW, RPW)])
        for q in range(NIDX):
            idx_start(q, q)
        idx_wait(0, 0)
        idx_wait(1, 1)
        g_start(0, 0)
        g_start(1, 1)
        plsc.subcore_barrier()

        @pl.loop(0, my_cpw, step=NIDX)
        def _(i):
            for q in range(NIDX):
                c = i + q
                r = q % NBUF
                g_wait(q, r)
                pltpu.sync_copy(rows[r], acc_sh.at[idx[q].at[1]], add=True)

                @pl.when(c + NBUF < my_cpw)
                def _():
                    idx_wait(c + NBUF, (q + NBUF) % NIDX)
                    g_start((q + NBUF) % NIDX, r)

                @pl.when(c + NIDX < my_cpw)
                def _():
                    idx_start(c + NIDX, q)

        plsc.subcore_barrier()
        pltpu.sync_copy(acc_sh.at[pl.ds(sid * RPW, RPW)],
                        out_hbm.at[cid, pl.ds(sid * RPW, RPW)])

    return _scatter_kernel


def _mm_body(x_ref, w_ref, o_ref):
    o_ref[...] = jnp.dot(x_ref[...], w_ref[...],
                         preferred_element_type=jnp.float32)


def _tc_matmul(x, w):
    n, k = x.shape
    m = w.shape[1]
    return pl.pallas_call(
        _mm_body,
        grid=(n // BLK,),
        in_specs=[pl.BlockSpec((BLK, k), lambda i: (i, 0)),
                  pl.BlockSpec((k, m), lambda i: (0, 0))],
        out_specs=pl.BlockSpec((BLK, m), lambda i: (i, 0)),
        out_shape=jax.ShapeDtypeStruct((n, m), jnp.float32),
    )(x, w)


def _prep_body(degp_ref, h1_ref, dinv_ref, g1_ref):
    deg = degp_ref[0] + degp_ref[1] + 1.0
    dinv = lax.rsqrt(deg)
    dinv_ref[...] = dinv
    g1_ref[...] = h1_ref[...] * dinv


def _tc_prep(degp, h1):
    return pl.pallas_call(
        _prep_body,
        grid=(NPAD // BLK,),
        in_specs=[pl.BlockSpec((2, BLK, D), lambda i: (0, i, 0)),
                  pl.BlockSpec((BLK, D), lambda i: (i, 0))],
        out_specs=[pl.BlockSpec((BLK, D), lambda i: (i, 0)),
                   pl.BlockSpec((BLK, D), lambda i: (i, 0))],
        out_shape=[jax.ShapeDtypeStruct((NPAD, D), jnp.float32),
                   jax.ShapeDtypeStruct((NPAD, D), jnp.float32)],
    )(degp, h1)


def _mid_body(s_ref, g1_ref, dinv_ref, b1_ref, wlin_ref, blin_ref, w2_ref,
              g2_ref):
    dinv = dinv_ref[...]
    s = s_ref[0] + s_ref[1] + g1_ref[...]
    a = s * dinv + b1_ref[...]
    t = jnp.tanh(a)
    l = jnp.dot(t, wlin_ref[...], preferred_element_type=jnp.float32)
    l = l + blin_ref[...]
    h2 = jnp.dot(l, w2_ref[...], preferred_element_type=jnp.float32)
    g2_ref[...] = h2 * dinv


def _tc_mid(s1, g1, dinv16, b1r, Wlin, blinr, W2):
    return pl.pallas_call(
        _mid_body,
        grid=(NPAD // BLK,),
        in_specs=[pl.BlockSpec((2, BLK, D), lambda i: (0, i, 0)),
                  pl.BlockSpec((BLK, D), lambda i: (i, 0)),
                  pl.BlockSpec((BLK, D), lambda i: (i, 0)),
                  pl.BlockSpec((1, D), lambda i: (0, 0)),
                  pl.BlockSpec((D, D), lambda i: (0, 0)),
                  pl.BlockSpec((1, D), lambda i: (0, 0)),
                  pl.BlockSpec((D, D), lambda i: (0, 0))],
        out_specs=pl.BlockSpec((BLK, D), lambda i: (i, 0)),
        out_shape=jax.ShapeDtypeStruct((NPAD, D), jnp.float32),
    )(s1, g1, dinv16, b1r, Wlin, blinr, W2)


def _fin_body(s_ref, g2_ref, dinv_ref, b2_ref, wout_ref, bout_ref, o_ref):
    i = pl.program_id(0)
    dinv = dinv_ref[...]
    a = (s_ref[0] + s_ref[1] + g2_ref[...]) * dinv + b2_ref[...]
    t = jnp.tanh(a)
    o = jnp.dot(t, wout_ref[...], preferred_element_type=jnp.float32)
    o = o + bout_ref[...]
    rows = lax.broadcasted_iota(jnp.int32, (BLK, OUTD), 0) + i * BLK
    o = jnp.where(rows < N_NODES, o, -jnp.inf)
    m = jnp.max(o, axis=0, keepdims=True)

    @pl.when(i == 0)
    def _():
        o_ref[...] = m

    @pl.when(i != 0)
    def _():
        o_ref[...] = jnp.maximum(o_ref[...], m)


def _tc_final(s2, g2, dinv16, b2r, Wout, boutr):
    return pl.pallas_call(
        _fin_body,
        grid=(NPAD // BLK,),
        in_specs=[pl.BlockSpec((2, BLK, D), lambda i: (0, i, 0)),
                  pl.BlockSpec((BLK, D), lambda i: (i, 0)),
                  pl.BlockSpec((BLK, D), lambda i: (i, 0)),
                  pl.BlockSpec((1, D), lambda i: (0, 0)),
                  pl.BlockSpec((D, OUTD), lambda i: (0, 0)),
                  pl.BlockSpec((1, OUTD), lambda i: (0, 0))],
        out_specs=pl.BlockSpec((1, OUTD), lambda i: (0, 0)),
        out_shape=jax.ShapeDtypeStruct((1, OUTD), jnp.float32),
    )(s2, g2, dinv16, b2r, Wout, boutr)


def kernel(x, edge_index, W1, b1, Wlin, blin, W2, b2, Wout, bout):
    src = edge_index[0].astype(jnp.int32)
    dst = edge_index[1].astype(jnp.int32)
    src_c = jnp.pad(src, (0, EPAD - E)).reshape(NCHUNKS, CHUNK)
    dst_c = jnp.pad(dst, (0, EPAD - E),
                    constant_values=N_NODES).reshape(NCHUNKS, CHUNK)
    eidx = jnp.stack([src_c, dst_c], axis=1).reshape(2 * NCHUNKS, CHUNK)
    x_p = jnp.pad(x, ((0, NPAD - N_NODES), (0, 0)))
    zerosD = jnp.zeros((RPW, D), jnp.float32)
    onesD = jnp.ones((CHUNK, D), jnp.float32)
    b1r = b1.reshape(1, D)
    blinr = blin.reshape(1, D)
    b2r = b2.reshape(1, D)
    boutr = bout.reshape(1, OUTD)

    deg_kernel = _get_deg_kernel()
    scatter_kernel = _get_scatter_kernel()
    degp = deg_kernel(eidx, onesD, zerosD)
    h1 = _tc_matmul(x_p, W1)
    dinv, g1 = _tc_prep(degp, h1)
    s1 = scatter_kernel(g1, eidx, zerosD)
    g2 = _tc_mid(s1, g1, dinv, b1r, Wlin, blinr, W2)
    s2 = scatter_kernel(g2, eidx, zerosD)
    return _tc_final(s2, g2, dinv, b2r, Wout, boutr)


# speedup vs baseline: 1.3193x; 1.1205x over previous
import functools

import jax
import jax.numpy as jnp
from jax import lax
from jax.experimental import pallas as pl
from jax.experimental.pallas import tpu as pltpu
from jax.experimental.pallas import tpu_sc as plsc

N_NODES = 10000
D = 128
OUTD = 64
E = 320000

NPAD = 10240
CHUNK = 128
NW = 32
CPW = 80
NCHUNKS = NW * CPW
EPAD = NCHUNKS * CHUNK
RPW = NPAD // 16
NBUF = 2
NIDX = 4
CPW0 = 132
CPW1 = NCHUNKS // 16 - CPW0
BLK = 1024


def _sc_compiler_params():
    import dataclasses
    cp = pltpu.CompilerParams()
    if "needs_layout_passes" in pltpu.CompilerParams.__dataclass_fields__:
        cp = dataclasses.replace(cp, needs_layout_passes=False)
    return cp


EPW = EPAD // 32
V = 16


@functools.cache
def _get_deg_kernel():
    mesh = plsc.VectorSubcoreMesh(core_axis_name="c", subcore_axis_name="s")

    @functools.partial(
        pl.kernel,
        out_type=jax.ShapeDtypeStruct((32, NPAD), jnp.float32),
        mesh=mesh,
        scratch_types=[
            pltpu.VMEM((EPW,), jnp.int32),
            pltpu.VMEM((NPAD,), jnp.float32),
        ],
        compiler_params=_sc_compiler_params(),
    )
    def _deg_kernel(dst_hbm, out_hbm, dstb, degrow):
        cid = lax.axis_index("c")
        sid = lax.axis_index("s")
        w = cid * 16 + sid
        pltpu.sync_copy(dst_hbm.at[pl.ds(w * EPW, EPW)], dstb)

        @pl.loop(0, NPAD, step=V)
        def _(i):
            degrow[pl.ds(i, V)] = jnp.zeros((V,), jnp.float32)

        ones = jnp.full((V,), 1.0, jnp.float32)

        @pl.loop(0, EPW, step=V)
        def _(j):
            d16 = dstb[pl.ds(j, V)]
            plsc.addupdate_scatter(degrow, [d16], ones)

        pltpu.sync_copy(degrow, out_hbm.at[w])

    return _deg_kernel


@functools.cache
def _get_scatter_kernel():
    mesh = plsc.VectorSubcoreMesh(core_axis_name="c", subcore_axis_name="s")

    @functools.partial(
        pl.kernel,
        out_type=jax.ShapeDtypeStruct((2, NPAD, D), jnp.float32),
        mesh=mesh,
        scratch_types=(
            [pltpu.VMEM((2, CHUNK), jnp.int32)] * NIDX
            + [pltpu.VMEM((CHUNK, D), jnp.float32)] * NBUF
            + [pltpu.VMEM_SHARED((NPAD, D), jnp.float32)]
            + [pltpu.SemaphoreType.DMA] * (NIDX + NBUF)
        ),
    )
    def _scatter_kernel(g_hbm, eidx_hbm, zeros_hbm, out_hbm, *rest):
        idx = rest[:NIDX]
        rows = rest[NIDX:NIDX + NBUF]
        acc_sh = rest[NIDX + NBUF]
        sem_i = rest[NIDX + NBUF + 1:NIDX + NBUF + 1 + NIDX]
        sem_g = rest[NIDX + NBUF + 1 + NIDX:]
        cid = lax.axis_index("c")
        sid = lax.axis_index("s")
        my_cpw = jnp.where(cid == 0, CPW0, CPW1)
        base = jnp.where(cid == 0, sid * CPW0, 16 * CPW0 + sid * CPW1)

        def idx_start(c, q):
            pltpu.async_copy(eidx_hbm.at[pl.ds(2 * (base + c), 2)], idx[q],
                             sem_i[q])

        def idx_wait(c, q):
            pltpu.make_async_copy(eidx_hbm.at[pl.ds(2 * (base + c), 2)],
                                  idx[q], sem_i[q]).wait()

        def g_start(q, r):
            pltpu.async_copy(g_hbm.at[idx[q].at[0]], rows[r], sem_g[r])

        def g_wait(q, r):
            pltpu.make_async_copy(g_hbm.at[idx[q].at[0]], rows[r],
                                  sem_g[r]).wait()

        pltpu.sync_copy(zeros_hbm, acc_sh.at[pl.ds(sid * RPW, RPW)])
        for q in range(NIDX):
            idx_start(q, q)
        idx_wait(0, 0)
        idx_wait(1, 1)
        g_start(0, 0)
        g_start(1, 1)
        plsc.subcore_barrier()

        @pl.loop(0, my_cpw, step=NIDX)
        def _(i):
            for q in range(NIDX):
                c = i + q
                r = q % NBUF
                g_wait(q, r)
                pltpu.sync_copy(rows[r], acc_sh.at[idx[q].at[1]], add=True)

                @pl.when(c + NBUF < my_cpw)
                def _():
                    idx_wait(c + NBUF, (q + NBUF) % NIDX)
                    g_start((q + NBUF) % NIDX, r)

                @pl.when(c + NIDX < my_cpw)
                def _():
                    idx_start(c + NIDX, q)

        plsc.subcore_barrier()
        pltpu.sync_copy(acc_sh.at[pl.ds(sid * RPW, RPW)],
                        out_hbm.at[cid, pl.ds(sid * RPW, RPW)])

    return _scatter_kernel


def _mm_body(x_ref, w_ref, o_ref):
    o_ref[...] = jnp.dot(x_ref[...], w_ref[...],
                         preferred_element_type=jnp.float32)


def _tc_matmul(x, w):
    n, k = x.shape
    m = w.shape[1]
    return pl.pallas_call(
        _mm_body,
        grid=(n // BLK,),
        in_specs=[pl.BlockSpec((BLK, k), lambda i: (i, 0)),
                  pl.BlockSpec((k, m), lambda i: (0, 0))],
        out_specs=pl.BlockSpec((BLK, m), lambda i: (i, 0)),
        out_shape=jax.ShapeDtypeStruct((n, m), jnp.float32),
    )(x, w)


def _prep_body(degp_ref, h1_ref, dinv_ref, g1_ref):
    ones32 = jnp.ones((32, 1), jnp.float32)
    deg = lax.dot_general(degp_ref[...], ones32,
                          dimension_numbers=(((0,), (0,)), ((), ())),
                          preferred_element_type=jnp.float32)
    dinv = lax.rsqrt(deg + 1.0)
    dinv_ref[...] = dinv
    g1_ref[...] = h1_ref[...] * dinv


def _tc_prep(degp, h1):
    return pl.pallas_call(
        _prep_body,
        grid=(NPAD // BLK,),
        in_specs=[pl.BlockSpec((32, BLK), lambda i: (0, i)),
                  pl.BlockSpec((BLK, D), lambda i: (i, 0))],
        out_specs=[pl.BlockSpec((BLK, 1), lambda i: (i, 0)),
                   pl.BlockSpec((BLK, D), lambda i: (i, 0))],
        out_shape=[jax.ShapeDtypeStruct((NPAD, 1), jnp.float32),
                   jax.ShapeDtypeStruct((NPAD, D), jnp.float32)],
    )(degp, h1)


def _mid_body(s_ref, g1_ref, dinv_ref, b1_ref, wlin_ref, blin_ref, w2_ref,
              g2_ref):
    dinv = dinv_ref[...]
    s = s_ref[0] + s_ref[1] + g1_ref[...]
    a = s * dinv + b1_ref[...]
    t = jnp.tanh(a)
    l = jnp.dot(t, wlin_ref[...], preferred_element_type=jnp.float32)
    l = l + blin_ref[...]
    h2 = jnp.dot(l, w2_ref[...], preferred_element_type=jnp.float32)
    g2_ref[...] = h2 * dinv


def _tc_mid(s1, g1, dinv16, b1r, Wlin, blinr, W2):
    return pl.pallas_call(
        _mid_body,
        grid=(NPAD // BLK,),
        in_specs=[pl.BlockSpec((2, BLK, D), lambda i: (0, i, 0)),
                  pl.BlockSpec((BLK, D), lambda i: (i, 0)),
                  pl.BlockSpec((BLK, 1), lambda i: (i, 0)),
                  pl.BlockSpec((1, D), lambda i: (0, 0)),
                  pl.BlockSpec((D, D), lambda i: (0, 0)),
                  pl.BlockSpec((1, D), lambda i: (0, 0)),
                  pl.BlockSpec((D, D), lambda i: (0, 0))],
        out_specs=pl.BlockSpec((BLK, D), lambda i: (i, 0)),
        out_shape=jax.ShapeDtypeStruct((NPAD, D), jnp.float32),
    )(s1, g1, dinv16, b1r, Wlin, blinr, W2)


def _fin_body(s_ref, g2_ref, dinv_ref, b2_ref, wout_ref, bout_ref, o_ref):
    i = pl.program_id(0)
    dinv = dinv_ref[...]
    a = (s_ref[0] + s_ref[1] + g2_ref[...]) * dinv + b2_ref[...]
    t = jnp.tanh(a)
    o = jnp.dot(t, wout_ref[...], preferred_element_type=jnp.float32)
    o = o + bout_ref[...]
    rows = lax.broadcasted_iota(jnp.int32, (BLK, OUTD), 0) + i * BLK
    o = jnp.where(rows < N_NODES, o, -jnp.inf)
    m = jnp.max(o, axis=0, keepdims=True)

    @pl.when(i == 0)
    def _():
        o_ref[...] = m

    @pl.when(i != 0)
    def _():
        o_ref[...] = jnp.maximum(o_ref[...], m)


def _tc_final(s2, g2, dinv16, b2r, Wout, boutr):
    return pl.pallas_call(
        _fin_body,
        grid=(NPAD // BLK,),
        in_specs=[pl.BlockSpec((2, BLK, D), lambda i: (0, i, 0)),
                  pl.BlockSpec((BLK, D), lambda i: (i, 0)),
                  pl.BlockSpec((BLK, 1), lambda i: (i, 0)),
                  pl.BlockSpec((1, D), lambda i: (0, 0)),
                  pl.BlockSpec((D, OUTD), lambda i: (0, 0)),
                  pl.BlockSpec((1, OUTD), lambda i: (0, 0))],
        out_specs=pl.BlockSpec((1, OUTD), lambda i: (0, 0)),
        out_shape=jax.ShapeDtypeStruct((1, OUTD), jnp.float32),
    )(s2, g2, dinv16, b2r, Wout, boutr)


def kernel(x, edge_index, W1, b1, Wlin, blin, W2, b2, Wout, bout):
    src = edge_index[0].astype(jnp.int32)
    dst = edge_index[1].astype(jnp.int32)
    src_f = jnp.pad(src, (0, EPAD - E))
    dst_f = jnp.pad(dst, (0, EPAD - E), constant_values=N_NODES)
    src_c = src_f.reshape(NCHUNKS, CHUNK)
    dst_c = dst_f.reshape(NCHUNKS, CHUNK)
    eidx = jnp.stack([src_c, dst_c], axis=1).reshape(2 * NCHUNKS, CHUNK)
    x_p = jnp.pad(x, ((0, NPAD - N_NODES), (0, 0)))
    zerosD = jnp.zeros((RPW, D), jnp.float32)
    b1r = b1.reshape(1, D)
    blinr = blin.reshape(1, D)
    b2r = b2.reshape(1, D)
    boutr = bout.reshape(1, OUTD)

    deg_kernel = _get_deg_kernel()
    scatter_kernel = _get_scatter_kernel()
    degp = deg_kernel(dst_f)
    h1 = _tc_matmul(x_p, W1)
    dinv, g1 = _tc_prep(degp, h1)
    s1 = scatter_kernel(g1, eidx, zerosD)
    g2 = _tc_mid(s1, g1, dinv, b1r, Wlin, blinr, W2)
    s2 = scatter_kernel(g2, eidx, zerosD)
    return _tc_final(s2, g2, dinv, b2r, Wout, boutr)
